# baseline (speedup 1.0000x reference)
import functools

import jax
import jax.numpy as jnp
from jax import lax
from jax.experimental import pallas as pl
from jax.experimental.pallas import tpu as pltpu
from jax.experimental.pallas import tpu_sc as plsc

_B = 16384
_D = 64
_LY = 10
_LN = 20
_Z = 32
_L = 16

_NC = 2
_NS = 16
_NW = _NC * _NS
_RPW = _B // _NW
_CB = 8
_NCH = _RPW // _CB
_MASKED_Z = 100.0


def _sc_logits(W_in, W_out, bX, bA):
    mesh = plsc.VectorSubcoreMesh(core_axis_name="c", subcore_axis_name="s")

    @functools.partial(
        pl.kernel,
        out_type=jax.ShapeDtypeStruct((_NW, _RPW * _Z), jnp.float32),
        mesh=mesh,
        compiler_params=pltpu.CompilerParams(
            needs_layout_passes=False, use_tc_tiling_on_sc=False),
        scratch_types=[
            pltpu.VMEM((_RPW,), jnp.int32),
            pltpu.VMEM((_RPW * _Z,), jnp.int32),
            pltpu.VMEM((_CB, _D), jnp.float32),
            pltpu.VMEM((_CB, _D), jnp.float32),
            pltpu.VMEM((_CB * _Z, _D), jnp.float32),
            pltpu.VMEM((_CB * _Z, _D), jnp.float32),
            pltpu.VMEM((_L, _L), jnp.float32),
            pltpu.VMEM((_RPW * _Z,), jnp.float32),
            pltpu.SemaphoreType.DMA,
            pltpu.SemaphoreType.DMA,
        ],
    )
    def k(w_in, w_out, bx, ba, out,
          ix_v, ia_v, xr0, xr1, ar0, ar1, t_v, z_v, sem0, sem1):
        wid = lax.axis_index("s") * _NC + lax.axis_index("c")
        lanes = lax.iota(jnp.int32, _L)
        sign0 = jnp.where(lanes < _LY, 1.0, -1.0)

        pltpu.sync_copy(bx.at[wid], ix_v)
        pltpu.sync_copy(ba.at[wid], ia_v)

        def gather(c, xr, ar, sem):
            pltpu.async_copy(
                w_in.at[ix_v.at[pl.ds(c * _CB, _CB)]], xr, sem)
            pltpu.async_copy(
                w_out.at[ia_v.at[pl.ds(c * _CB * _Z, _CB * _Z)]], ar, sem)

        def drain(c, xr, ar, sem):
            pltpu.make_async_copy(
                w_in.at[ix_v.at[pl.ds(c * _CB, _CB)]], xr, sem).wait()
            pltpu.make_async_copy(
                w_out.at[ia_v.at[pl.ds(c * _CB * _Z, _CB * _Z)]], ar,
                sem).wait()

        def compute(c, xr, ar):
            def row_body(i, carry2):
                xs = [xr[i, pl.ds(16 * t, 16)] for t in range(_D // 16)]
                for g in range(2):
                    base = i * _Z + g * _L
                    for j in range(_L):
                        r = base + j
                        acc = xs[0] * ar[r, pl.ds(0, 16)]
                        for t in range(1, _D // 16):
                            acc += xs[t] * ar[r, pl.ds(16 * t, 16)]
                        plsc.store_scatter(
                            t_v, [lanes, jnp.full((_L,), j, jnp.int32)], acc)
                    z = t_v[0]
                    for d in range(1, _L):
                        z = z + t_v[d]
                    gbase = c * _CB * _Z + base
                    if g == 0:
                        idxv = ia_v[pl.ds(gbase, _L)]
                        z = jnp.where((lanes < _LY) & (idxv == 0),
                                      _MASKED_Z, z * sign0)
                    else:
                        z = jnp.where(lanes >= _L - 2, _MASKED_Z, -z)
                    z_v[pl.ds(gbase, _L)] = z
                return carry2

            lax.fori_loop(0, _CB, row_body, None)

        gather(0, xr0, ar0, sem0)

        def outer(cc, carry):
            c0 = 2 * cc
            gather(c0 + 1, xr1, ar1, sem1)
            drain(c0, xr0, ar0, sem0)
            compute(c0, xr0, ar0)

            @pl.when(cc < _NCH // 2 - 1)
            def _():
                gather(c0 + 2, xr0, ar0, sem0)

            drain(c0 + 1, xr1, ar1, sem1)
            compute(c0 + 1, xr1, ar1)
            return carry

        lax.fori_loop(0, _NCH // 2, outer, None)
        pltpu.sync_copy(z_v, out.at[wid])

    return k(W_in, W_out, bX, bA)


def _tc_loss(z):
    def body(z_ref, o_ref):
        t = -z_ref[...]
        sp = jnp.maximum(t, 0.0) + jnp.log1p(jnp.exp(-jnp.abs(t)))
        o_ref[0, 0] = jnp.sum(sp) * (1.0 / _B)

    out = pl.pallas_call(
        body,
        out_shape=jax.ShapeDtypeStruct((1, 1), jnp.float32),
        out_specs=pl.BlockSpec(memory_space=pltpu.SMEM),
    )(z)
    return out[0, 0]


def kernel(batch_X, batch_Y, batch_N, W_in, W_out):
    pad = (jnp.arange(_B, dtype=jnp.int32) % jnp.int32(W_out.shape[0]))
    pad = jnp.broadcast_to(pad[:, None], (_B, _Z - _LY - _LN))
    bA = jnp.concatenate([batch_Y, batch_N, pad], axis=1)
    bX = batch_X.reshape(_NW, _RPW)
    bA = bA.reshape(_NW, _RPW * _Z)
    z = _sc_logits(W_in, W_out, bX, bA)
    return _tc_loss(z.reshape(_B, _Z))


# speedup vs baseline: 1.0087x; 1.0087x over previous
import functools

import jax
import jax.numpy as jnp
from jax import lax
from jax.experimental import pallas as pl
from jax.experimental.pallas import tpu as pltpu
from jax.experimental.pallas import tpu_sc as plsc

_B = 16384
_D = 64
_LY = 10
_LN = 20
_Z = 32
_L = 16

_NC = 2
_NS = 16
_NW = _NC * _NS
_RPW = _B // _NW
_CB = 8
_NCH = _RPW // _CB
_MASKED_Z = 100.0


def _sc_logits(W_in, W_out, bX, bA):
    mesh = plsc.VectorSubcoreMesh(core_axis_name="c", subcore_axis_name="s")

    @functools.partial(
        pl.kernel,
        out_type=jax.ShapeDtypeStruct((_NW, _RPW * _Z), jnp.float32),
        mesh=mesh,
        compiler_params=pltpu.CompilerParams(
            needs_layout_passes=False, use_tc_tiling_on_sc=False),
        scratch_types=[
            pltpu.VMEM((_RPW,), jnp.int32),
            pltpu.VMEM((_RPW * _Z,), jnp.int32),
            pltpu.VMEM((_CB, _D), jnp.float32),
            pltpu.VMEM((_CB, _D), jnp.float32),
            pltpu.VMEM((_CB * _Z, _D), jnp.float32),
            pltpu.VMEM((_CB * _Z, _D), jnp.float32),
            pltpu.VMEM((_L, _L), jnp.float32),
            pltpu.VMEM((_RPW * _Z,), jnp.float32),
            pltpu.SemaphoreType.DMA,
            pltpu.SemaphoreType.DMA,
        ],
    )
    def k(w_in, w_out, bx, ba, out,
          ix_v, ia_v, xr0, xr1, ar0, ar1, t_v, z_v, sem0, sem1):
        wid = lax.axis_index("s") * _NC + lax.axis_index("c")
        lanes = lax.iota(jnp.int32, _L)
        sign0 = jnp.where(lanes < _LY, 1.0, -1.0)

        pltpu.sync_copy(bx.at[wid], ix_v)
        pltpu.sync_copy(ba.at[wid], ia_v)

        def gather(c, xr, ar, sem):
            pltpu.async_copy(
                w_in.at[ix_v.at[pl.ds(c * _CB, _CB)]], xr, sem)
            pltpu.async_copy(
                w_out.at[ia_v.at[pl.ds(c * _CB * _Z, _CB * _Z)]], ar, sem)

        def drain(c, xr, ar, sem):
            pltpu.make_async_copy(
                w_in.at[ix_v.at[pl.ds(c * _CB, _CB)]], xr, sem).wait()
            pltpu.make_async_copy(
                w_out.at[ia_v.at[pl.ds(c * _CB * _Z, _CB * _Z)]], ar,
                sem).wait()

        def compute(c, xr, ar):
            def row_body(i, carry2):
                xs = [xr[i, pl.ds(16 * t, 16)] for t in range(_D // 16)]
                for g in range(2):
                    base = i * _Z + g * _L
                    for j in range(_L):
                        r = base + j
                        acc = xs[0] * ar[r, pl.ds(0, 16)]
                        for t in range(1, _D // 16):
                            acc += xs[t] * ar[r, pl.ds(16 * t, 16)]
                        plsc.store_scatter(
                            t_v, [lanes, jnp.full((_L,), j, jnp.int32)], acc)
                    z = t_v[0]
                    for d in range(1, _L):
                        z = z + t_v[d]
                    gbase = c * _CB * _Z + base
                    if g == 0:
                        idxv = ia_v[pl.ds(gbase, _L)]
                        z = jnp.where((lanes < _LY) & (idxv == 0),
                                      _MASKED_Z, z * sign0)
                    else:
                        z = jnp.where(lanes >= _L - 2, _MASKED_Z, -z)
                    z_v[pl.ds(gbase, _L)] = z
                return carry2

            lax.fori_loop(0, _CB, row_body, None)

        gather(0, xr0, ar0, sem0)

        def outer(cc, carry):
            c0 = 2 * cc
            gather(c0 + 1, xr1, ar1, sem1)
            drain(c0, xr0, ar0, sem0)
            compute(c0, xr0, ar0)

            @pl.when(cc < _NCH // 2 - 1)
            def _():
                gather(c0 + 2, xr0, ar0, sem0)

            drain(c0 + 1, xr1, ar1, sem1)
            compute(c0 + 1, xr1, ar1)
            return carry

        lax.fori_loop(0, _NCH // 2, outer, None)
        pltpu.sync_copy(z_v, out.at[wid])

    return k(W_in, W_out, bX, bA)


def _tc_loss(z):
    def body(z_ref, o_ref):
        t = -z_ref[...]
        sp = jnp.maximum(t, 0.0) + jnp.log1p(jnp.exp(-jnp.abs(t)))
        o_ref[0, 0] = jnp.sum(sp) * (1.0 / _B)

    out = pl.pallas_call(
        body,
        out_shape=jax.ShapeDtypeStruct((1, 1), jnp.float32),
        out_specs=pl.BlockSpec(memory_space=pltpu.SMEM),
    )(z)
    return out[0, 0]


def kernel(batch_X, batch_Y, batch_N, W_in, W_out):
    pad = (jnp.arange(_B, dtype=jnp.int32) % jnp.int32(W_out.shape[0]))
    pad = jnp.broadcast_to(pad[:, None], (_B, _Z - _LY - _LN))
    bA = jnp.concatenate([batch_Y, batch_N, pad], axis=1)
    bX = batch_X.reshape(_NW, _RPW)
    bA = bA.reshape(_NW, _RPW * _Z)
    z = _sc_logits(W_in, W_out, bX, bA)
    return _tc_loss(z)
